# baseline (speedup 1.0000x reference)
import functools

import jax
import jax.numpy as jnp
from jax import lax
from jax.experimental import pallas as pl
from jax.experimental.pallas import tpu as pltpu
from jax.experimental.pallas import tpu_sc as plsc

L = 16


@functools.lru_cache(maxsize=None)
def _build(B, NEG, V, D):
    info = plsc.get_sparse_core_info()
    NC, NS = info.num_cores, info.num_subcores
    NW = NC * NS
    K = 2 + NEG
    CH = 8
    ROWS = CH * K
    D2 = 2 * D
    GE = 4
    GR = GE * K
    GF = GR * D
    WV = GF // L
    NG = CH // GE
    BW = B // NW
    n_chunks = BW // CH
    assert B % NW == 0 and n_chunks % 2 == 0 and V % 2 == 0 and GF % L == 0

    mesh = plsc.VectorSubcoreMesh(core_axis_name="c", subcore_axis_name="s")

    @functools.partial(
        pl.kernel,
        mesh=mesh,
        compiler_params=pltpu.CompilerParams(
            use_tc_tiling_on_sc=False, needs_layout_passes=False),
        out_type=jax.ShapeDtypeStruct((B * K * D // GF, GF), jnp.float32),
        scratch_types=[
            pltpu.VMEM((ROWS,), jnp.int32),
            pltpu.VMEM((ROWS,), jnp.int32),
            pltpu.VMEM((CH * 8,), jnp.int32),
            pltpu.VMEM((CH * 8,), jnp.int32),
            pltpu.VMEM((ROWS, D2), jnp.float32),
            pltpu.VMEM((ROWS, D2), jnp.float32),
            pltpu.VMEM((NG, GF), jnp.float32),
            pltpu.VMEM((NG, GF), jnp.float32),
            pltpu.SemaphoreType.DMA,
            pltpu.SemaphoreType.DMA,
            pltpu.SemaphoreType.DMA,
            pltpu.SemaphoreType.DMA,
        ],
    )
    def skipgram(gidx_hbm, bc_hbm, wt2_hbm, wc2_hbm, out_hbm,
                 gidx0, gidx1, bc0, bc1, pairs0, pairs1, outb0, outb1,
                 gsem0, gsem1, wsem0, wsem1):
        wid = lax.axis_index("s") * NC + lax.axis_index("c")
        iota = lax.iota(jnp.int32, L)

        def do_chunk(g, c, gidx_v, bc_v, pairs, outb, gsem, wsem):
            gci = wid * n_chunks + c
            row0 = gci * ROWS

            pltpu.sync_copy(gidx_hbm.at[pl.ds(row0, ROWS)], gidx_v)
            pltpu.sync_copy(bc_hbm.at[pl.ds(gci * (CH * 8), CH * 8)], bc_v)

            g_t = pltpu.make_async_copy(
                wt2_hbm.at[gidx_v.at[pl.ds(0, CH)]],
                pairs.at[pl.ds(0, CH)], gsem)
            g_c = pltpu.make_async_copy(
                wc2_hbm.at[gidx_v.at[pl.ds(CH, ROWS - CH)]],
                pairs.at[pl.ds(CH, ROWS - CH)], gsem)
            g_t.start()
            g_c.start()

            @pl.when(g >= 1)
            def _():
                pltpu.make_async_copy(
                    outb, out_hbm.at[pl.ds(gci * NG, NG)], wsem).wait()

            g_t.wait()
            g_c.wait()

            def group(i4, carry):
                off32 = pl.multiple_of(i4 * (GE * 8), 8)
                ext0 = bc_v[pl.ds(off32, L)]
                ext1 = bc_v[pl.ds(off32 + L, L)]

                rowinfo = []
                for rr in range(GR):
                    e, j = rr // K, rr % K
                    i_el = i4 * GE + e
                    if j == 0:
                        srcrow = i_el
                    elif j == 1:
                        srcrow = CH + i_el
                    else:
                        srcrow = 2 * CH + i_el * NEG + (j - 2)
                    base = (ext0 if e < 2 else ext1)[(e % 2) * 8 + j]
                    rowinfo.append((jnp.broadcast_to(srcrow, (L,)).astype(jnp.int32),
                                    base))

                for w in range(WV):
                    p0 = w * L
                    rr = p0 // D
                    off = p0 - rr * D
                    rowv0, base0 = rowinfo[rr]
                    if off + L <= D:
                        colv = jnp.broadcast_to(base0 + off, (L,)) + iota
                        x = plsc.load_gather(pairs, [rowv0, colv])
                    else:
                        cut = D - off
                        rowv1, base1 = rowinfo[rr + 1]
                        msk = iota < cut
                        rowv = jnp.where(msk, rowv0, rowv1)
                        colv = jnp.where(
                            msk,
                            jnp.broadcast_to(base0 + off, (L,)),
                            jnp.broadcast_to(base1 - cut, (L,))) + iota
                        x = plsc.load_gather(pairs, [rowv, colv])
                    outb[i4, pl.ds(p0, L)] = x
                return carry

            lax.fori_loop(0, NG, group, 0)

            pltpu.make_async_copy(
                outb, out_hbm.at[pl.ds(gci * NG, NG)], wsem).start()

        def loop_body(g, carry):
            do_chunk(g, 2 * g, gidx0, bc0, pairs0, outb0, gsem0, wsem0)
            do_chunk(g, 2 * g + 1, gidx1, bc1, pairs1, outb1, gsem1, wsem1)
            return carry

        lax.fori_loop(0, n_chunks // 2, loop_body, 0)

        pltpu.make_async_copy(outb0, out_hbm.at[pl.ds(0, NG)], wsem0).wait()
        pltpu.make_async_copy(outb1, out_hbm.at[pl.ds(0, NG)], wsem1).wait()

    return skipgram


def kernel(target_words, context_words, negative_examples, W_target, W_context):
    B = target_words.shape[0]
    NEG = negative_examples.shape[1]
    V, D = W_target.shape
    K = 2 + NEG
    CH = 8
    tw = target_words.astype(jnp.int32)
    cw = context_words.astype(jnp.int32)
    ne = negative_examples.astype(jnp.int32)

    gidx = jnp.concatenate(
        [(tw >> 1).reshape(B // CH, CH),
         (cw >> 1).reshape(B // CH, CH),
         (ne >> 1).reshape(B // CH, CH * NEG)], axis=1).reshape(B * K)
    comb = jnp.concatenate([tw[:, None], cw[:, None], ne], axis=1)
    bc = (comb & 1) * D
    bc = jnp.concatenate([bc, jnp.zeros((B, 1), jnp.int32)], axis=1)
    bc = bc.reshape(B * 8)

    wt2 = W_target.reshape(V // 2, 2 * D)
    wc2 = W_context.reshape(V // 2, 2 * D)
    fn = _build(B, NEG, V, D)
    out = fn(gidx, bc, wt2, wc2)
    return out.reshape(B, K, D)


# speedup vs baseline: 1.2894x; 1.2894x over previous
import functools

import jax
import jax.numpy as jnp
from jax import lax
from jax.experimental import pallas as pl
from jax.experimental.pallas import tpu as pltpu
from jax.experimental.pallas import tpu_sc as plsc

L = 16
NSEM = 4


@functools.lru_cache(maxsize=None)
def _build(B, NEG, V, D):
    info = plsc.get_sparse_core_info()
    NC, NS = info.num_cores, info.num_subcores
    NW = NC * NS
    K = 2 + NEG
    CH = 16
    ROWS = CH * K
    BW = B // NW
    n_chunks = BW // CH
    RPS = ROWS // NSEM
    assert B % NW == 0 and BW % (2 * CH) == 0 and ROWS % NSEM == 0

    mesh = plsc.VectorSubcoreMesh(core_axis_name="c", subcore_axis_name="s")

    @functools.partial(
        pl.kernel,
        mesh=mesh,
        compiler_params=pltpu.CompilerParams(
            use_tc_tiling_on_sc=False, needs_layout_passes=False),
        out_type=jax.ShapeDtypeStruct((B * K, D), jnp.float32),
        scratch_types=[
            pltpu.VMEM((BW * K,), jnp.int32),
            pltpu.VMEM((ROWS, D), jnp.float32),
            pltpu.VMEM((ROWS, D), jnp.float32),
            pltpu.SemaphoreType.DMA((2, NSEM)),
            pltpu.SemaphoreType.DMA,
            pltpu.SemaphoreType.DMA,
        ],
    )
    def skipgram(comb_hbm, wt_hbm, wc_hbm, out_hbm,
                 idx_w, buf0, buf1, gsems, wsem0, wsem1):
        wid = lax.axis_index("s") * NC + lax.axis_index("c")
        wrow0 = wid * (BW * K)

        pltpu.sync_copy(comb_hbm.at[pl.ds(wrow0, BW * K)], idx_w)

        def do_chunk(g, c, sb, buf, wsem):
            row0 = wrow0 + c * ROWS
            lb = pl.multiple_of(c * ROWS, 8)

            @pl.when(g >= 1)
            def _():
                pltpu.make_async_copy(
                    buf, out_hbm.at[pl.ds(row0, ROWS)], wsem).wait()

            for v in range(ROWS // L):
                vec = idx_w[pl.ds(lb + v * L, L)]
                for j in range(L):
                    k = v * L + j
                    src = wt_hbm if k % K == 0 else wc_hbm
                    pltpu.make_async_copy(
                        src.at[pl.ds(vec[j], 1)],
                        buf.at[pl.ds(k, 1)],
                        gsems.at[sb, k % NSEM]).start()

            for q in range(NSEM):
                pltpu.make_async_copy(
                    wt_hbm.at[pl.ds(0, RPS)],
                    buf.at[pl.ds(q * RPS, RPS)],
                    gsems.at[sb, q]).wait()

            pltpu.make_async_copy(
                buf, out_hbm.at[pl.ds(row0, ROWS)], wsem).start()

        def loop_body(g, carry):
            do_chunk(g, 2 * g, 0, buf0, wsem0)
            do_chunk(g, 2 * g + 1, 1, buf1, wsem1)
            return carry

        lax.fori_loop(0, n_chunks // 2, loop_body, 0)

        pltpu.make_async_copy(
            buf0, out_hbm.at[pl.ds(wrow0, ROWS)], wsem0).wait()
        pltpu.make_async_copy(
            buf1, out_hbm.at[pl.ds(wrow0, ROWS)], wsem1).wait()

    return skipgram


def kernel(target_words, context_words, negative_examples, W_target, W_context):
    B = target_words.shape[0]
    NEG = negative_examples.shape[1]
    V, D = W_target.shape
    tw = target_words.astype(jnp.int32)
    cw = context_words.astype(jnp.int32)
    ne = negative_examples.astype(jnp.int32)
    K = 2 + NEG
    comb = jnp.concatenate([tw[:, None], cw[:, None], ne], axis=1).reshape(B * K)
    fn = _build(B, NEG, V, D)
    out = fn(comb, W_target, W_context)
    return out.reshape(B, K, D)
